# baseline (speedup 1.0000x reference)
import jax
import jax.numpy as jnp
from jax import lax
from jax.experimental import pallas as pl
from jax.experimental.pallas import tpu as pltpu
from jax.experimental.pallas import tpu_sc as plsc

_M = 0.4
_GAMMA = 80.0
_NEG_BIG = -1e30
_NC, _NS, _L = 2, 16, 16
_NW = _NC * _NS
_N = 256
_D = 128
_NA = _N // 4
_NG = 8
_APG = 8
_NH = 4
_CPH = _N // _NH // _L
_SW = 6 * _L


def _sc_body(et_hbm, e_hbm, lab_hbm, stats_hbm,
             et_v, e_v, lab_v, stats_v, sem):
    wid = lax.axis_index("s") * _NC + lax.axis_index("c")
    g = wid // _NH
    h = wid % _NH
    h_et = pltpu.async_copy(et_hbm.at[h], et_v, sem)
    pltpu.sync_copy(lab_hbm, lab_v)
    pltpu.sync_copy(e_hbm.at[pl.ds(g * 32, 32)], e_v)
    lab_blk0 = lab_v[pl.ds(g * 32, _L)]
    lab_blk1 = lab_v[pl.ds(g * 32 + _L, _L)]
    lab_is = tuple(lab_blk0[4 * t] for t in range(4)) + \
        tuple(lab_blk1[4 * t] for t in range(4))
    h_et.wait()

    iota = lax.iota(jnp.int32, _L)
    zero = jnp.zeros((_L,), jnp.float32)

    def qstep(q, accs):
        cts = [e_v[4 * t, pl.ds(q * _L, _L)] for t in range(_APG)]
        accs = list(accs)
        for l in range(_L):
            bts = [cts[t][l] for t in range(_APG)]
            d = q * _L + l
            for c in range(_CPH):
                row = et_v[d, pl.ds(c * _L, _L)]
                for t in range(_APG):
                    accs[t * _CPH + c] = accs[t * _CPH + c] + bts[t] * row
        return tuple(accs)

    accs = lax.fori_loop(0, _D // _L, qstep, (zero,) * (_APG * _CPH))

    izero = jnp.zeros((_L,), jnp.int32)
    for t in range(_APG):
        i = g * 32 + 4 * t
        lab_i = izero + lab_is[t]
        i_vec = izero + i
        mlp = jnp.full((_L,), _NEG_BIG, jnp.float32)
        mln = jnp.full((_L,), _NEG_BIG, jnp.float32)
        slp, sln, cp, cn = zero, zero, zero, zero
        for c in range(_CPH):
            s = accs[t * _CPH + c]
            cc = h * _CPH + c
            labc = lab_v[pl.ds(cc * _L, _L)]
            col = iota + (cc * _L)
            same01 = jnp.where(labc == lab_i, 1.0, 0.0)
            ne01 = jnp.where(col == i_vec, 0.0, 1.0)
            posf = same01 * ne01
            negf = 1.0 - same01
            alpha_p = jnp.maximum((1.0 + _M) - s, 0.0)
            alpha_n = jnp.maximum(s + _M, 0.0)
            lp = (posf * (-_GAMMA * alpha_p * (s - (1.0 - _M)))
                  + (1.0 - posf) * _NEG_BIG)
            ln_ = (negf * (_GAMMA * alpha_n * (s - _M))
                   + (1.0 - negf) * _NEG_BIG)
            m2 = jnp.maximum(mlp, lp)
            slp = slp * jnp.exp(mlp - m2) + jnp.exp(lp - m2)
            mlp = m2
            m2 = jnp.maximum(mln, ln_)
            sln = sln * jnp.exp(mln - m2) + jnp.exp(ln_ - m2)
            mln = m2
            cp = cp + posf
            cn = cn + negf
        stats_v[t, pl.ds(0 * _L, _L)] = mlp
        stats_v[t, pl.ds(1 * _L, _L)] = slp
        stats_v[t, pl.ds(2 * _L, _L)] = mln
        stats_v[t, pl.ds(3 * _L, _L)] = sln
        stats_v[t, pl.ds(4 * _L, _L)] = cp
        stats_v[t, pl.ds(5 * _L, _L)] = cn

    pltpu.sync_copy(stats_v, stats_hbm.at[h, pl.ds(_APG * g, _APG)])


def _finalize_body(x_ref, bs_ref, out_ref):
    bs = bs_ref[0]

    def quarter_stats(hh):
        x = x_ref[hh]
        mlp = x[:, 0 * _L:1 * _L]
        slp = x[:, 1 * _L:2 * _L]
        mln = x[:, 2 * _L:3 * _L]
        sln = x[:, 3 * _L:4 * _L]
        cp = x[:, 4 * _L:5 * _L]
        cn = x[:, 5 * _L:6 * _L]
        mp = jnp.max(mlp, axis=1, keepdims=True)
        sp = jnp.sum(slp * jnp.exp(mlp - mp), axis=1, keepdims=True)
        mn = jnp.max(mln, axis=1, keepdims=True)
        sn = jnp.sum(sln * jnp.exp(mln - mn), axis=1, keepdims=True)
        return (mp, sp, mn, sn,
                jnp.sum(cp, axis=1, keepdims=True),
                jnp.sum(cn, axis=1, keepdims=True))

    def combine(a, b):
        mpa, spa, mna, sna, cpa, cna = a
        mpb, spb, mnb, snb, cpb, cnb = b
        mp = jnp.maximum(mpa, mpb)
        sp = spa * jnp.exp(mpa - mp) + spb * jnp.exp(mpb - mp)
        mn = jnp.maximum(mna, mnb)
        sn = sna * jnp.exp(mna - mn) + snb * jnp.exp(mnb - mn)
        return (mp, sp, mn, sn, cpa + cpb, cna + cnb)

    s01 = combine(quarter_stats(0), quarter_stats(1))
    s23 = combine(quarter_stats(2), quarter_stats(3))
    mp, sp, mn, sn, cp, cn = combine(s01, s23)

    ar = lax.broadcasted_iota(jnp.int32, (_NA, 1), 0) * 4
    filt = ((ar % 4 == 0) & (ar < bs)) | (ar > bs)
    valid = filt & (cp > 0) & (cn > 0)
    lse = mp + jnp.log(sp) + jnp.log(cn) + mn + jnp.log(sn) + jnp.log(cp)
    term = jnp.where(
        valid,
        jnp.maximum(lse, 0.0) + jnp.log1p(jnp.exp(-jnp.abs(lse))),
        0.0,
    )
    total = jnp.sum(term)
    cnt = jnp.sum(valid.astype(jnp.float32))
    out_ref[...] = jnp.where(cnt > 0, total / cnt, 0.0).reshape(1, 1)


def kernel(embeddings, labels, batch_size):
    e = embeddings.astype(jnp.float32)
    et = jnp.transpose(e.reshape(_NH, _N // _NH, _D), (0, 2, 1))
    lab = labels.astype(jnp.int32)
    mesh = plsc.VectorSubcoreMesh(
        core_axis_name="c", subcore_axis_name="s",
        num_cores=_NC, num_subcores=_NS,
    )
    stats = pl.kernel(
        _sc_body,
        out_type=jax.ShapeDtypeStruct((_NH, _NA, _SW), jnp.float32),
        mesh=mesh,
        scratch_types=[
            pltpu.VMEM((_D, _N // _NH), jnp.float32),
            pltpu.VMEM((32, _D), jnp.float32),
            pltpu.VMEM((_N,), jnp.int32),
            pltpu.VMEM((_APG, _SW), jnp.float32),
            pltpu.SemaphoreType.DMA,
        ],
    )(et, e, lab)

    bs = jnp.asarray(batch_size, jnp.int32).reshape(1)
    out = pl.pallas_call(
        _finalize_body,
        in_specs=[
            pl.BlockSpec(memory_space=pltpu.VMEM),
            pl.BlockSpec(memory_space=pltpu.SMEM),
        ],
        out_shape=jax.ShapeDtypeStruct((1, 1), jnp.float32),
    )(stats, bs)
    return out[0, 0]


# speedup vs baseline: 1.0026x; 1.0026x over previous
import jax
import jax.numpy as jnp
from jax import lax
from jax.experimental import pallas as pl
from jax.experimental.pallas import tpu as pltpu
from jax.experimental.pallas import tpu_sc as plsc

_M = 0.4
_GAMMA = 80.0
_NEG_BIG = -1e30
_NC, _NS, _L = 2, 16, 16
_NW = _NC * _NS
_N = 256
_D = 128
_NA = _N // 4
_NG = 8
_APG = 8
_NH = 4
_CPH = _N // _NH // _L
_SW = 6 * _L


def _sc_body(et_hbm, e_hbm, lab_hbm, stats_hbm,
             et_v, e_v, lab_v, stats_v, sem):
    wid = lax.axis_index("s") * _NC + lax.axis_index("c")
    g = wid // _NH
    h = wid % _NH
    h_et = pltpu.async_copy(et_hbm.at[h], et_v, sem)
    pltpu.sync_copy(lab_hbm, lab_v)
    pltpu.sync_copy(e_hbm.at[pl.ds(g * 32, 32)], e_v)
    lab_blk0 = lab_v[pl.ds(g * 32, _L)]
    lab_blk1 = lab_v[pl.ds(g * 32 + _L, _L)]
    lab_is = tuple(lab_blk0[4 * t] for t in range(4)) + \
        tuple(lab_blk1[4 * t] for t in range(4))
    h_et.wait()

    iota = lax.iota(jnp.int32, _L)
    zero = jnp.zeros((_L,), jnp.float32)

    @plsc.parallel_loop(0, _D // _L, unroll=2, carry=(zero,) * (_APG * _CPH))
    def qstep(q, accs):
        cts = [e_v[4 * t, pl.ds(q * _L, _L)] for t in range(_APG)]
        accs = list(accs)
        for l in range(_L):
            bts = [cts[t][l] for t in range(_APG)]
            d = q * _L + l
            for c in range(_CPH):
                row = et_v[d, pl.ds(c * _L, _L)]
                for t in range(_APG):
                    accs[t * _CPH + c] = accs[t * _CPH + c] + bts[t] * row
        return tuple(accs)

    accs = qstep

    izero = jnp.zeros((_L,), jnp.int32)
    for t in range(_APG):
        i = g * 32 + 4 * t
        lab_i = izero + lab_is[t]
        i_vec = izero + i
        mlp = jnp.full((_L,), _NEG_BIG, jnp.float32)
        mln = jnp.full((_L,), _NEG_BIG, jnp.float32)
        slp, sln, cp, cn = zero, zero, zero, zero
        for c in range(_CPH):
            s = accs[t * _CPH + c]
            cc = h * _CPH + c
            labc = lab_v[pl.ds(cc * _L, _L)]
            col = iota + (cc * _L)
            same01 = jnp.where(labc == lab_i, 1.0, 0.0)
            ne01 = jnp.where(col == i_vec, 0.0, 1.0)
            posf = same01 * ne01
            negf = 1.0 - same01
            alpha_p = jnp.maximum((1.0 + _M) - s, 0.0)
            alpha_n = jnp.maximum(s + _M, 0.0)
            lp = (posf * (-_GAMMA * alpha_p * (s - (1.0 - _M)))
                  + (1.0 - posf) * _NEG_BIG)
            ln_ = (negf * (_GAMMA * alpha_n * (s - _M))
                   + (1.0 - negf) * _NEG_BIG)
            m2 = jnp.maximum(mlp, lp)
            slp = slp * jnp.exp(mlp - m2) + jnp.exp(lp - m2)
            mlp = m2
            m2 = jnp.maximum(mln, ln_)
            sln = sln * jnp.exp(mln - m2) + jnp.exp(ln_ - m2)
            mln = m2
            cp = cp + posf
            cn = cn + negf
        stats_v[t, pl.ds(0 * _L, _L)] = mlp
        stats_v[t, pl.ds(1 * _L, _L)] = slp
        stats_v[t, pl.ds(2 * _L, _L)] = mln
        stats_v[t, pl.ds(3 * _L, _L)] = sln
        stats_v[t, pl.ds(4 * _L, _L)] = cp
        stats_v[t, pl.ds(5 * _L, _L)] = cn

    pltpu.sync_copy(stats_v, stats_hbm.at[h, pl.ds(_APG * g, _APG)])


def _finalize_body(x_ref, bs_ref, out_ref):
    bs = bs_ref[0]

    def quarter_stats(hh):
        x = x_ref[hh]
        mlp = x[:, 0 * _L:1 * _L]
        slp = x[:, 1 * _L:2 * _L]
        mln = x[:, 2 * _L:3 * _L]
        sln = x[:, 3 * _L:4 * _L]
        cp = x[:, 4 * _L:5 * _L]
        cn = x[:, 5 * _L:6 * _L]
        mp = jnp.max(mlp, axis=1, keepdims=True)
        sp = jnp.sum(slp * jnp.exp(mlp - mp), axis=1, keepdims=True)
        mn = jnp.max(mln, axis=1, keepdims=True)
        sn = jnp.sum(sln * jnp.exp(mln - mn), axis=1, keepdims=True)
        return (mp, sp, mn, sn,
                jnp.sum(cp, axis=1, keepdims=True),
                jnp.sum(cn, axis=1, keepdims=True))

    def combine(a, b):
        mpa, spa, mna, sna, cpa, cna = a
        mpb, spb, mnb, snb, cpb, cnb = b
        mp = jnp.maximum(mpa, mpb)
        sp = spa * jnp.exp(mpa - mp) + spb * jnp.exp(mpb - mp)
        mn = jnp.maximum(mna, mnb)
        sn = sna * jnp.exp(mna - mn) + snb * jnp.exp(mnb - mn)
        return (mp, sp, mn, sn, cpa + cpb, cna + cnb)

    s01 = combine(quarter_stats(0), quarter_stats(1))
    s23 = combine(quarter_stats(2), quarter_stats(3))
    mp, sp, mn, sn, cp, cn = combine(s01, s23)

    ar = lax.broadcasted_iota(jnp.int32, (_NA, 1), 0) * 4
    filt = ((ar % 4 == 0) & (ar < bs)) | (ar > bs)
    valid = filt & (cp > 0) & (cn > 0)
    lse = mp + jnp.log(sp) + jnp.log(cn) + mn + jnp.log(sn) + jnp.log(cp)
    term = jnp.where(
        valid,
        jnp.maximum(lse, 0.0) + jnp.log1p(jnp.exp(-jnp.abs(lse))),
        0.0,
    )
    total = jnp.sum(term)
    cnt = jnp.sum(valid.astype(jnp.float32))
    out_ref[...] = jnp.where(cnt > 0, total / cnt, 0.0).reshape(1, 1)


def kernel(embeddings, labels, batch_size):
    e = embeddings.astype(jnp.float32)
    et = jnp.transpose(e.reshape(_NH, _N // _NH, _D), (0, 2, 1))
    lab = labels.astype(jnp.int32)
    mesh = plsc.VectorSubcoreMesh(
        core_axis_name="c", subcore_axis_name="s",
        num_cores=_NC, num_subcores=_NS,
    )
    stats = pl.kernel(
        _sc_body,
        out_type=jax.ShapeDtypeStruct((_NH, _NA, _SW), jnp.float32),
        mesh=mesh,
        scratch_types=[
            pltpu.VMEM((_D, _N // _NH), jnp.float32),
            pltpu.VMEM((32, _D), jnp.float32),
            pltpu.VMEM((_N,), jnp.int32),
            pltpu.VMEM((_APG, _SW), jnp.float32),
            pltpu.SemaphoreType.DMA,
        ],
    )(et, e, lab)

    bs = jnp.asarray(batch_size, jnp.int32).reshape(1)
    out = pl.pallas_call(
        _finalize_body,
        in_specs=[
            pl.BlockSpec(memory_space=pltpu.VMEM),
            pl.BlockSpec(memory_space=pltpu.SMEM),
        ],
        out_shape=jax.ShapeDtypeStruct((1, 1), jnp.float32),
    )(stats, bs)
    return out[0, 0]
